# Initial kernel scaffold; baseline (speedup 1.0000x reference)
import jax
import jax.numpy as jnp
from jax.experimental import pallas as pl


def kernel(inputs, embeddings):
    raise NotImplementedError("write your pallas kernel here")


# speedup vs baseline: 1.1491x; 1.1491x over previous
import functools

import jax
import jax.numpy as jnp
from jax import lax
from jax.experimental import pallas as pl
from jax.experimental.pallas import tpu as pltpu
from jax.experimental.pallas import tpu_sc as plsc

NUM_EMB = 8192
DIM = 64
N_TOK = 16384
TM = 128

NC = 2
NS = 16
NW = NC * NS
B_PER_W = N_TOK // NW
IDX_CHUNK = 128
N_CHUNK = B_PER_W // IDX_CHUNK


def _dist_argmin_body(x_ref, et_ref, rowsum_ref, colsum_ref, idx_ref, dmin_ref):
    x = x_ref[...].astype(jnp.bfloat16)
    et = et_ref[...]
    mm = lax.dot_general(x, et, (((1,), (0,)), ((), ())),
                         preferred_element_type=jnp.float32)
    d = (rowsum_ref[...] + colsum_ref[...]) - 2.0 * mm
    half = NUM_EMB // 2
    d1 = d[:, :half]
    d2 = d[:, half:]
    iota = lax.broadcasted_iota(jnp.int32, (TM, half), 1)
    m1 = jnp.min(d1, axis=1, keepdims=True)
    i1 = jnp.min(jnp.where(d1 == m1, iota, half), axis=1, keepdims=True)
    m2 = jnp.min(d2, axis=1, keepdims=True)
    i2 = jnp.min(jnp.where(d2 == m2, iota, half), axis=1, keepdims=True) + half
    m1b = m1.astype(jnp.bfloat16).astype(jnp.float32)
    pick2 = m2 < m1b
    idx_ref[...] = jnp.where(pick2, i2, i1)
    dmin_ref[...] = jnp.where(pick2, m2, m1)


def _dist_argmin(flat, e_t, rowsum, colsum):
    nb = N_TOK // TM
    return pl.pallas_call(
        _dist_argmin_body,
        grid=(nb,),
        in_specs=[
            pl.BlockSpec((TM, DIM), lambda i: (i, 0)),
            pl.BlockSpec((DIM, NUM_EMB), lambda i: (0, 0)),
            pl.BlockSpec((TM, 1), lambda i: (i, 0)),
            pl.BlockSpec((1, NUM_EMB), lambda i: (0, 0)),
        ],
        out_specs=[
            pl.BlockSpec((TM, 1), lambda i: (i, 0)),
            pl.BlockSpec((TM, 1), lambda i: (i, 0)),
        ],
        out_shape=[
            jax.ShapeDtypeStruct((N_TOK, 1), jnp.int32),
            jax.ShapeDtypeStruct((N_TOK, 1), jnp.float32),
        ],
    )(flat, e_t, rowsum, colsum)


def _sc_gather_body(table_hbm, idx_hbm, x_hbm, dmin_hbm, out_hbm, partial_hbm,
                    idx_v, rows_v, x_v, dmin_v, acc_v, sem0, sem1):
    wid = lax.axis_index("s") * NC + lax.axis_index("c")
    base = wid * B_PER_W
    sems = (sem0, sem1)

    pltpu.sync_copy(idx_hbm.at[wid], idx_v)
    pltpu.sync_copy(x_hbm.at[pl.ds(base, B_PER_W)], x_v)
    pltpu.sync_copy(dmin_hbm.at[wid], dmin_v)

    def gather(j):
        return pltpu.async_copy(table_hbm.at[idx_v.at[j]],
                                rows_v.at[j % 2], sems[j % 2])

    cp = gather(0)
    for j in range(N_CHUNK):
        nxt = gather(j + 1) if j + 1 < N_CHUNK else None
        cp.wait()

        def row_body(r, carry, j=j):
            for c in range(DIM // 16):
                s = pl.ds(c * 16, 16)
                q = rows_v[j % 2, r, s]
                xx = x_v[j * IDX_CHUNK + r, s]
                x_v[j * IDX_CHUNK + r, s] = xx + (q - xx)
            return carry

        lax.fori_loop(0, IDX_CHUNK, row_body, 0, unroll=False)
        cp = nxt

    pltpu.sync_copy(x_v, out_hbm.at[pl.ds(base, B_PER_W)])

    def sum_body(k, acc):
        return acc + dmin_v[pl.ds(k * 16, 16)]

    acc = lax.fori_loop(0, B_PER_W // 16, sum_body,
                        jnp.zeros((16,), jnp.float32), unroll=False)
    acc_v[...] = acc
    pltpu.sync_copy(acc_v, partial_hbm.at[wid])


@functools.partial(jax.jit, static_argnums=())
def _sc_gather(embeddings, idx_flat, flat, dmin_flat):
    mesh = plsc.VectorSubcoreMesh(core_axis_name="c", subcore_axis_name="s")
    f = pl.kernel(
        _sc_gather_body,
        mesh=mesh,
        out_type=[
            jax.ShapeDtypeStruct((N_TOK, DIM), jnp.float32),
            jax.ShapeDtypeStruct((NW, 16), jnp.float32),
        ],
        scratch_types=[
            pltpu.VMEM((N_CHUNK, IDX_CHUNK), jnp.int32),
            pltpu.VMEM((2, IDX_CHUNK, 128), jnp.float32),
            pltpu.VMEM((B_PER_W, DIM), jnp.float32),
            pltpu.VMEM((B_PER_W,), jnp.float32),
            pltpu.VMEM((16,), jnp.float32),
            pltpu.SemaphoreType.DMA,
            pltpu.SemaphoreType.DMA,
        ],
    )
    table = jnp.pad(embeddings, ((0, 0), (0, 128 - DIM)))
    idx3 = idx_flat.reshape(NW, N_CHUNK, IDX_CHUNK)
    dmin2 = dmin_flat.reshape(NW, B_PER_W)
    return f(table, idx3, flat, dmin2)


def kernel(inputs, embeddings):
    flat = inputs.reshape(N_TOK, DIM)
    e_t = embeddings.T
    rowsum = jnp.sum(flat ** 2, axis=1, keepdims=True)
    colsum = jnp.sum(embeddings ** 2, axis=1)[None, :]

    idx2d, dmin2d = _dist_argmin(flat, e_t, rowsum, colsum)

    out_st, partials = _sc_gather(embeddings, idx2d.reshape(N_TOK),
                                  flat, dmin2d.reshape(N_TOK))

    m = jnp.sum(partials) / (N_TOK * DIM)
    loss = m + 0.25 * m
    quantized = out_st.reshape(inputs.shape)
    return (loss, quantized, idx2d)
